# baseline (speedup 1.0000x reference)
import jax
import jax.numpy as jnp
from jax.experimental import pallas as pl
from jax.experimental.pallas import tpu as pltpu

_NT = 4096
_D = 4096
_E = 64
_CAP = 64
_TBLK = 256
_GRID = _NT // _TBLK


def _router_kernel(x_ref, w_ref, comb_ref, disp_ref, laux_ref, cnt_ref, gsum_ref):
    step = pl.program_id(0)

    @pl.when(step == 0)
    def _():
        cnt_ref[...] = jnp.zeros_like(cnt_ref)
        gsum_ref[...] = jnp.zeros_like(gsum_ref)

    x = x_ref[...]
    w = w_ref[...]
    logits = jax.lax.dot_general(
        x.astype(jnp.bfloat16), w.astype(jnp.bfloat16), (((1,), (1,)), ((), ())),
        preferred_element_type=jnp.float32)
    m = jnp.max(logits, axis=1, keepdims=True)
    ex = jnp.exp(logits - m)
    den = jnp.sum(ex, axis=1, keepdims=True)
    gates = ex / den

    gmax = jnp.max(gates, axis=1, keepdims=True)
    iota_e = jax.lax.broadcasted_iota(jnp.int32, (_TBLK, _E), 1)
    idx = jnp.min(jnp.where(gates == gmax, iota_e, _E), axis=1, keepdims=True)
    maskf = (iota_e == idx).astype(jnp.float32)

    r = jax.lax.broadcasted_iota(jnp.int32, (_TBLK, _TBLK), 0)
    c = jax.lax.broadcasted_iota(jnp.int32, (_TBLK, _TBLK), 1)
    tri = (c <= r).astype(jnp.bfloat16)
    cum = jax.lax.dot_general(
        tri, maskf.astype(jnp.bfloat16), (((1,), (0,)), ((), ())),
        preferred_element_type=jnp.float32)

    prev = cnt_ref[...]
    loc = prev + cum - 1.0
    loc_own = jnp.sum(loc * maskf, axis=1, keepdims=True)
    keep = loc_own < float(_CAP)

    iota_e3 = jax.lax.broadcasted_iota(jnp.int32, (_TBLK, _E, _CAP), 1)
    iota_c3 = jax.lax.broadcasted_iota(jnp.int32, (_TBLK, _E, _CAP), 2)
    loc_i = loc_own.astype(jnp.int32)
    hit3 = ((iota_e3 == idx[:, :, None]) & (iota_c3 == loc_i[:, :, None])
            & keep[:, :, None])
    comb_ref[...] = jnp.where(hit3, gmax[:, :, None], jnp.float32(0.0))
    disp_ref[...] = hit3

    cnt_ref[...] = prev + cum[_TBLK - 1:_TBLK, :]
    gsum_ref[...] = gsum_ref[...] + jnp.sum(gates, axis=0, keepdims=True)
    laux = (jnp.float32(_E) / (_NT * _NT)) * jnp.sum(
        cnt_ref[...] * gsum_ref[...])
    laux_ref[...] = jnp.reshape(laux, (1, 1))


@jax.jit
def kernel(input, W):
    comb, disp, laux = pl.pallas_call(
        _router_kernel,
        grid=(_GRID,),
        in_specs=[
            pl.BlockSpec((_TBLK, _D), lambda i: (i, 0)),
            pl.BlockSpec((_E, _D), lambda i: (0, 0)),
        ],
        out_specs=[
            pl.BlockSpec((_TBLK, _E, _CAP), lambda i: (i, 0, 0)),
            pl.BlockSpec((_TBLK, _E, _CAP), lambda i: (i, 0, 0)),
            pl.BlockSpec((1, 1), lambda i: (0, 0)),
        ],
        out_shape=[
            jax.ShapeDtypeStruct((_NT, _E, _CAP), jnp.float32),
            jax.ShapeDtypeStruct((_NT, _E, _CAP), jnp.bool_),
            jax.ShapeDtypeStruct((1, 1), jnp.float32),
        ],
        scratch_shapes=[
            pltpu.VMEM((1, _E), jnp.float32),
            pltpu.VMEM((1, _E), jnp.float32),
        ],
        compiler_params=pltpu.CompilerParams(
            dimension_semantics=("arbitrary",)),
    )(input, W)
    return laux[0, 0], comb, disp


# speedup vs baseline: 3.5225x; 3.5225x over previous
import jax
import jax.numpy as jnp
from jax.experimental import pallas as pl
from jax.experimental.pallas import tpu as pltpu

_NT = 4096
_D = 4096
_E = 64
_CAP = 64
_TBLK = 256
_GRID = _NT // _TBLK


def _router_kernel(x_ref, w_ref, comb_ref, disp_ref, laux_ref, cnt_ref, gsum_ref):
    step = pl.program_id(0)

    @pl.when(step == 0)
    def _():
        cnt_ref[...] = jnp.zeros_like(cnt_ref)
        gsum_ref[...] = jnp.zeros_like(gsum_ref)

    x = x_ref[...]
    w = w_ref[...]
    logits_te = jax.lax.dot_general(
        x.astype(jnp.bfloat16), w.astype(jnp.bfloat16), (((1,), (1,)), ((), ())),
        preferred_element_type=jnp.float32)
    logits = jnp.transpose(logits_te)

    m = jnp.max(logits, axis=0, keepdims=True)
    ex = jnp.exp(logits - m)
    den = jnp.sum(ex, axis=0, keepdims=True)
    gates = ex / den

    gmax = jnp.max(gates, axis=0, keepdims=True)
    iota_e = jax.lax.broadcasted_iota(jnp.int32, (_E, _TBLK), 0)
    idx = jnp.min(jnp.where(gates == gmax, iota_e, _E), axis=0, keepdims=True)
    maskf = (iota_e == idx).astype(jnp.float32)

    r = jax.lax.broadcasted_iota(jnp.int32, (_TBLK, _TBLK), 0)
    c = jax.lax.broadcasted_iota(jnp.int32, (_TBLK, _TBLK), 1)
    triu = (r <= c).astype(jnp.bfloat16)
    cum = jax.lax.dot_general(
        maskf.astype(jnp.bfloat16), triu, (((1,), (0,)), ((), ())),
        preferred_element_type=jnp.float32)

    prev = cnt_ref[...]
    loc = prev + cum - 1.0
    loc_own = jnp.sum(loc * maskf, axis=0, keepdims=True)
    keep = loc_own < float(_CAP)
    loc_i = loc_own.astype(jnp.int32)

    iota_e3 = jax.lax.broadcasted_iota(jnp.int32, (_E, _CAP, _TBLK), 0)
    iota_c3 = jax.lax.broadcasted_iota(jnp.int32, (_E, _CAP, _TBLK), 1)
    hit3 = ((iota_e3 == idx[:, None, :]) & (iota_c3 == loc_i[:, None, :])
            & keep[:, None, :])
    comb_ref[...] = jnp.where(hit3, gmax[:, None, :], jnp.float32(0.0))
    disp_ref[...] = hit3.astype(jnp.int8)

    cnt_ref[...] = prev + cum[:, _TBLK - 1:_TBLK]
    gsum_ref[...] = gsum_ref[...] + jnp.sum(gates, axis=1, keepdims=True)
    laux = (jnp.float32(_E) / (_NT * _NT)) * jnp.sum(
        cnt_ref[...] * gsum_ref[...])
    laux_ref[...] = jnp.reshape(laux, (1, 1))


@jax.jit
def kernel(input, W):
    comb, disp, laux = pl.pallas_call(
        _router_kernel,
        grid=(_GRID,),
        in_specs=[
            pl.BlockSpec((_TBLK, _D), lambda i: (i, 0)),
            pl.BlockSpec((_E, _D), lambda i: (0, 0)),
        ],
        out_specs=[
            pl.BlockSpec((_E, _CAP, _TBLK), lambda i: (0, 0, i)),
            pl.BlockSpec((_E, _CAP, _TBLK), lambda i: (0, 0, i)),
            pl.BlockSpec((1, 1), lambda i: (0, 0)),
        ],
        out_shape=[
            jax.ShapeDtypeStruct((_E, _CAP, _NT), jnp.float32),
            jax.ShapeDtypeStruct((_E, _CAP, _NT), jnp.int8),
            jax.ShapeDtypeStruct((1, 1), jnp.float32),
        ],
        scratch_shapes=[
            pltpu.VMEM((_E, 1), jnp.float32),
            pltpu.VMEM((_E, 1), jnp.float32),
        ],
        compiler_params=pltpu.CompilerParams(
            dimension_semantics=("arbitrary",)),
    )(input, W)
    combine = jnp.transpose(comb, (2, 0, 1))
    dispatch = jnp.transpose(disp, (2, 0, 1)).astype(jnp.bool_)
    return laux[0, 0], combine, dispatch


# speedup vs baseline: 3.9524x; 1.1221x over previous
import jax
import jax.numpy as jnp
from jax.experimental import pallas as pl
from jax.experimental.pallas import tpu as pltpu

_NT = 4096
_D = 4096
_E = 64
_CAP = 64
_TBLK = 256
_GRID = _NT // _TBLK


def _router_kernel(x_ref, w_ref, comb_ref, disp_ref, laux_ref, cnt_ref, gsum_ref):
    step = pl.program_id(0)

    @pl.when(step == 0)
    def _():
        cnt_ref[...] = jnp.zeros_like(cnt_ref)
        gsum_ref[...] = jnp.zeros_like(gsum_ref)

    x = x_ref[...]
    w = w_ref[...]
    logits_te = jax.lax.dot_general(
        x.astype(jnp.bfloat16), w.astype(jnp.bfloat16), (((1,), (1,)), ((), ())),
        preferred_element_type=jnp.float32)
    logits = jnp.transpose(logits_te)

    m = jnp.max(logits, axis=0, keepdims=True)
    ex = jnp.exp(logits - m)
    den = jnp.sum(ex, axis=0, keepdims=True)
    gates = ex / den

    gmax = jnp.max(gates, axis=0, keepdims=True)
    iota_e = jax.lax.broadcasted_iota(jnp.int32, (_E, _TBLK), 0)
    idx = jnp.min(jnp.where(gates == gmax, iota_e, _E), axis=0, keepdims=True)
    maskf = (iota_e == idx).astype(jnp.float32)

    r = jax.lax.broadcasted_iota(jnp.int32, (_TBLK, _TBLK), 0)
    c = jax.lax.broadcasted_iota(jnp.int32, (_TBLK, _TBLK), 1)
    triu = (r <= c).astype(jnp.bfloat16)
    cum = jax.lax.dot_general(
        maskf.astype(jnp.bfloat16), triu, (((1,), (0,)), ((), ())),
        preferred_element_type=jnp.float32)

    prev = cnt_ref[...]
    loc = prev + cum - 1.0
    loc_own = jnp.sum(loc * maskf, axis=0, keepdims=True)
    keep = loc_own < float(_CAP)
    loc_i = loc_own.astype(jnp.int32)

    iota_c2 = jax.lax.broadcasted_iota(jnp.int32, (_CAP, _TBLK), 0)
    eq_e = iota_e == idx
    slotg = jnp.where((iota_c2 == loc_i) & keep, gmax, jnp.float32(0.0))
    comb3 = eq_e[:, None, :].astype(jnp.float32) * slotg[None, :, :]
    comb_ref[...] = comb3
    disp_ref[...] = (comb3 != jnp.float32(0.0)).astype(jnp.int8)

    cnt_ref[...] = prev + cum[:, _TBLK - 1:_TBLK]
    gsum_ref[...] = gsum_ref[...] + jnp.sum(gates, axis=1, keepdims=True)
    laux = (jnp.float32(_E) / (_NT * _NT)) * jnp.sum(
        cnt_ref[...] * gsum_ref[...])
    laux_ref[...] = jnp.reshape(laux, (1, 1))


@jax.jit
def kernel(input, W):
    comb, disp, laux = pl.pallas_call(
        _router_kernel,
        grid=(_GRID,),
        in_specs=[
            pl.BlockSpec((_TBLK, _D), lambda i: (i, 0)),
            pl.BlockSpec((_E, _D), lambda i: (0, 0)),
        ],
        out_specs=[
            pl.BlockSpec((_E, _CAP, _TBLK), lambda i: (0, 0, i)),
            pl.BlockSpec((_E, _CAP, _TBLK), lambda i: (0, 0, i)),
            pl.BlockSpec((1, 1), lambda i: (0, 0)),
        ],
        out_shape=[
            jax.ShapeDtypeStruct((_E, _CAP, _NT), jnp.float32),
            jax.ShapeDtypeStruct((_E, _CAP, _NT), jnp.int8),
            jax.ShapeDtypeStruct((1, 1), jnp.float32),
        ],
        scratch_shapes=[
            pltpu.VMEM((_E, 1), jnp.float32),
            pltpu.VMEM((_E, 1), jnp.float32),
        ],
        compiler_params=pltpu.CompilerParams(
            dimension_semantics=("arbitrary",)),
    )(input, W)
    combine = jnp.transpose(comb, (2, 0, 1))
    dispatch = jnp.transpose(disp, (2, 0, 1)).astype(jnp.bool_)
    return laux[0, 0], combine, dispatch


# speedup vs baseline: 4.0594x; 1.0271x over previous
import jax
import jax.numpy as jnp
from jax.experimental import pallas as pl
from jax.experimental.pallas import tpu as pltpu

_NT = 4096
_D = 4096
_E = 64
_CAP = 64
_TBLK = 256
_GRID = _NT // _TBLK


def _router_kernel(x_ref, w_ref, comb_ref, disp_ref, laux_ref, cnt_ref, gsum_ref):
    step = pl.program_id(0)

    @pl.when(step == 0)
    def _():
        cnt_ref[...] = jnp.zeros_like(cnt_ref)
        gsum_ref[...] = jnp.zeros_like(gsum_ref)

    x = x_ref[...]
    w = w_ref[...]
    logits_te = jax.lax.dot_general(
        x.astype(jnp.bfloat16), w.astype(jnp.bfloat16), (((1,), (1,)), ((), ())),
        preferred_element_type=jnp.float32)
    logits = jnp.transpose(logits_te)

    m = jnp.max(logits, axis=0, keepdims=True)
    ex = jnp.exp(logits - m)
    den = jnp.sum(ex, axis=0, keepdims=True)
    gates = ex / den

    gmax = jnp.max(gates, axis=0, keepdims=True)
    iota_e = jax.lax.broadcasted_iota(jnp.int32, (_E, _TBLK), 0)
    idx = jnp.min(jnp.where(gates == gmax, iota_e, _E), axis=0, keepdims=True)
    maskf = (iota_e == idx).astype(jnp.float32)

    r = jax.lax.broadcasted_iota(jnp.int32, (_TBLK, _TBLK), 0)
    c = jax.lax.broadcasted_iota(jnp.int32, (_TBLK, _TBLK), 1)
    triu = (r <= c).astype(jnp.bfloat16)
    cum = jax.lax.dot_general(
        maskf.astype(jnp.bfloat16), triu, (((1,), (0,)), ((), ())),
        preferred_element_type=jnp.float32)

    prev = cnt_ref[...]
    loc = prev + cum - 1.0
    loc_own = jnp.sum(loc * maskf, axis=0, keepdims=True)
    keep = loc_own < float(_CAP)
    loc_i = loc_own.astype(jnp.int32)

    iota_c2 = jax.lax.broadcasted_iota(jnp.int32, (_CAP, _TBLK), 0)
    eq_e = iota_e == idx
    slotg = jnp.where((iota_c2 == loc_i) & keep, gmax, jnp.float32(0.0))
    comb3 = eq_e[:, None, :].astype(jnp.float32) * slotg[None, :, :]
    comb_ref[...] = comb3
    disp_ref[...] = (comb3 != jnp.float32(0.0)).astype(jnp.int8)

    cnt_ref[...] = prev + cum[:, _TBLK - 1:_TBLK]
    gsum_ref[...] = gsum_ref[...] + jnp.sum(gates, axis=1, keepdims=True)
    laux = (jnp.float32(_E) / (_NT * _NT)) * jnp.sum(
        cnt_ref[...] * gsum_ref[...])
    laux_ref[...] = jnp.reshape(laux, (1, 1))


@jax.jit
def kernel(input, W):
    comb, disp, laux = pl.pallas_call(
        _router_kernel,
        grid=(_GRID,),
        in_specs=[
            pl.BlockSpec((_TBLK, _D), lambda i: (i, 0)),
            pl.BlockSpec((_E, _D), lambda i: (0, 0)),
        ],
        out_specs=[
            pl.BlockSpec((_E, _CAP, _TBLK), lambda i: (0, 0, i)),
            pl.BlockSpec((_E, _CAP, _TBLK), lambda i: (0, 0, i)),
            pl.BlockSpec((1, 1), lambda i: (0, 0)),
        ],
        out_shape=[
            jax.ShapeDtypeStruct((_E, _CAP, _NT), jnp.float32),
            jax.ShapeDtypeStruct((_E, _CAP, _NT), jnp.int8),
            jax.ShapeDtypeStruct((1, 1), jnp.float32),
        ],
        scratch_shapes=[
            pltpu.VMEM((_E, 1), jnp.float32),
            pltpu.VMEM((_E, 1), jnp.float32),
        ],
        compiler_params=pltpu.CompilerParams(
            dimension_semantics=("arbitrary",)),
    )(input, W)
    combine = jnp.transpose(comb, (2, 0, 1))
    dispatch = jnp.transpose(disp != jnp.int8(0), (2, 0, 1))
    return laux[0, 0], combine, dispatch


# speedup vs baseline: 4.2602x; 1.0495x over previous
import jax
import jax.numpy as jnp
from jax.experimental import pallas as pl
from jax.experimental.pallas import tpu as pltpu

_NT = 4096
_D = 4096
_E = 64
_CAP = 64
_TBLK = 512
_GRID = _NT // _TBLK


def _router_kernel(x_ref, w_ref, comb_ref, disp_ref, laux_ref, cnt_ref, gsum_ref):
    step = pl.program_id(0)

    @pl.when(step == 0)
    def _():
        cnt_ref[...] = jnp.zeros_like(cnt_ref)
        gsum_ref[...] = jnp.zeros_like(gsum_ref)

    x = x_ref[...]
    w = w_ref[...]
    logits_te = jax.lax.dot_general(
        x.astype(jnp.bfloat16), w.astype(jnp.bfloat16), (((1,), (1,)), ((), ())),
        preferred_element_type=jnp.float32)
    logits = jnp.transpose(logits_te)

    m = jnp.max(logits, axis=0, keepdims=True)
    ex = jnp.exp(logits - m)
    den = jnp.sum(ex, axis=0, keepdims=True)
    gates = ex / den

    gmax = jnp.max(gates, axis=0, keepdims=True)
    iota_e = jax.lax.broadcasted_iota(jnp.int32, (_E, _TBLK), 0)
    idx = jnp.min(jnp.where(gates == gmax, iota_e, _E), axis=0, keepdims=True)
    maskf = (iota_e == idx).astype(jnp.float32)

    r = jax.lax.broadcasted_iota(jnp.int32, (_TBLK, _TBLK), 0)
    c = jax.lax.broadcasted_iota(jnp.int32, (_TBLK, _TBLK), 1)
    triu = (r <= c).astype(jnp.bfloat16)
    cum = jax.lax.dot_general(
        maskf.astype(jnp.bfloat16), triu, (((1,), (0,)), ((), ())),
        preferred_element_type=jnp.float32)

    prev = cnt_ref[...]
    loc = prev + cum - 1.0
    loc_own = jnp.sum(loc * maskf, axis=0, keepdims=True)
    keep = loc_own < float(_CAP)
    loc_i = loc_own.astype(jnp.int32)

    iota_c2 = jax.lax.broadcasted_iota(jnp.int32, (_CAP, _TBLK), 0)
    eq_e = iota_e == idx
    slotg = jnp.where((iota_c2 == loc_i) & keep, gmax, jnp.float32(0.0))
    comb3 = eq_e[:, None, :].astype(jnp.float32) * slotg[None, :, :]
    comb_ref[...] = comb3
    disp_ref[...] = (comb3 != jnp.float32(0.0)).astype(jnp.int8)

    cnt_ref[...] = prev + cum[:, _TBLK - 1:_TBLK]
    gsum_ref[...] = gsum_ref[...] + jnp.sum(gates, axis=1, keepdims=True)
    laux = (jnp.float32(_E) / (_NT * _NT)) * jnp.sum(
        cnt_ref[...] * gsum_ref[...])
    laux_ref[...] = jnp.reshape(laux, (1, 1))


@jax.jit
def kernel(input, W):
    comb, disp, laux = pl.pallas_call(
        _router_kernel,
        grid=(_GRID,),
        in_specs=[
            pl.BlockSpec((_TBLK, _D), lambda i: (i, 0)),
            pl.BlockSpec((_E, _D), lambda i: (0, 0)),
        ],
        out_specs=[
            pl.BlockSpec((_E, _CAP, _TBLK), lambda i: (0, 0, i)),
            pl.BlockSpec((_E, _CAP, _TBLK), lambda i: (0, 0, i)),
            pl.BlockSpec((1, 1), lambda i: (0, 0)),
        ],
        out_shape=[
            jax.ShapeDtypeStruct((_E, _CAP, _NT), jnp.float32),
            jax.ShapeDtypeStruct((_E, _CAP, _NT), jnp.int8),
            jax.ShapeDtypeStruct((1, 1), jnp.float32),
        ],
        scratch_shapes=[
            pltpu.VMEM((_E, 1), jnp.float32),
            pltpu.VMEM((_E, 1), jnp.float32),
        ],
        compiler_params=pltpu.CompilerParams(
            dimension_semantics=("arbitrary",)),
    )(input, W)
    combine = jnp.transpose(comb, (2, 0, 1))
    dispatch = jnp.transpose(disp != jnp.int8(0), (2, 0, 1))
    return laux[0, 0], combine, dispatch
